# baseline (speedup 1.0000x reference)
import functools

import jax
import jax.numpy as jnp
from jax.experimental import pallas as pl
from jax.experimental.pallas import tpu as pltpu

_F32 = jnp.float32


def _soft_assign_block(h, c_t):
    d2 = (
        jnp.sum(h * h, axis=1, keepdims=True)
        - 2.0 * jnp.dot(h, c_t, preferred_element_type=_F32)
        + jnp.sum(c_t * c_t, axis=0, keepdims=True)
    )
    q = 1.0 / (1.0 + d2)
    return q / jnp.sum(q, axis=1, keepdims=True)


def _mm1_body(a_ref, b_ref, o_ref, *, act):
    acc = jnp.dot(a_ref[...], b_ref[...], preferred_element_type=_F32)
    o_ref[...] = _apply_act(acc, act)


def _mm1_bias_body(a_ref, b_ref, bias_ref, o_ref, *, act):
    acc = jnp.dot(a_ref[...], b_ref[...], preferred_element_type=_F32)
    o_ref[...] = _apply_act(acc + bias_ref[...], act)


def _mmk_body(a_ref, b_ref, o_ref, acc_ref, *, nk, act):
    k = pl.program_id(1)

    @pl.when(k == 0)
    def _init():
        acc_ref[...] = jnp.zeros_like(acc_ref)

    acc_ref[...] += jnp.dot(a_ref[...], b_ref[...], preferred_element_type=_F32)

    @pl.when(k == nk - 1)
    def _fin():
        o_ref[...] = _apply_act(acc_ref[...], act)


def _apply_act(acc, act):
    if act == "relu":
        return jnp.maximum(acc, 0.0)
    if act == "sigmoid":
        return jax.nn.sigmoid(acc)
    if act == "p5":
        lane = jax.lax.broadcasted_iota(jnp.int32, acc.shape, 1)
        keep = (lane >= 128) & (lane < 256)
        return jnp.where(keep, acc, jnp.maximum(acc, 0.0))
    return acc


def _mm(a, b, act="none", bm=1024, bk=512, bias=None):
    m, k = a.shape
    n = b.shape[1]
    bm = min(bm, m)
    if k <= 2048:
        grid = (m // bm,)
        in_specs = [
            pl.BlockSpec((bm, k), lambda i: (i, 0)),
            pl.BlockSpec((k, n), lambda i: (0, 0)),
        ]
        operands = [a, b]
        if bias is not None:
            body = functools.partial(_mm1_bias_body, act=act)
            in_specs.append(pl.BlockSpec((1, n), lambda i: (0, 0)))
            operands.append(bias)
        else:
            body = functools.partial(_mm1_body, act=act)
        return pl.pallas_call(
            body,
            grid=grid,
            in_specs=in_specs,
            out_specs=pl.BlockSpec((bm, n), lambda i: (i, 0)),
            out_shape=jax.ShapeDtypeStruct((m, n), _F32),
            compiler_params=pltpu.CompilerParams(
                dimension_semantics=("parallel",),
            ),
        )(*operands)
    nk = k // bk
    return pl.pallas_call(
        functools.partial(_mmk_body, nk=nk, act=act),
        grid=(m // bm, nk),
        in_specs=[
            pl.BlockSpec((bm, bk), lambda i, j: (i, j)),
            pl.BlockSpec((bk, n), lambda i, j: (j, 0)),
        ],
        out_specs=pl.BlockSpec((bm, n), lambda i, j: (i, 0)),
        out_shape=jax.ShapeDtypeStruct((m, n), _F32),
        scratch_shapes=[pltpu.VMEM((bm, n), _F32)],
        compiler_params=pltpu.CompilerParams(
            dimension_semantics=("parallel", "arbitrary"),
        ),
    )(a, b)


def _ae_body(
    x_ref,
    e1w, e1b, e2w, e2b, e3w, e3b, zlw, zlb,
    d1w, d1b, d2w, d2b, d3w, d3b, xbw, xbb,
    ct_ref,
    xbar_o, r_o, re1_o, q1_o,
):
    def lin(t, w, b):
        return jnp.dot(t, w[...], preferred_element_type=_F32) + b[...]

    x = x_ref[...]
    re1 = jnp.maximum(lin(x, e1w, e1b), 0.0)
    re2 = jnp.maximum(lin(re1, e2w, e2b), 0.0)
    re3 = jnp.maximum(lin(re2, e3w, e3b), 0.0)
    r = lin(re3, zlw, zlb)
    rd1 = jnp.maximum(lin(r, d1w, d1b), 0.0)
    rd2 = jnp.maximum(lin(rd1, d2w, d2b), 0.0)
    rd3 = jnp.maximum(lin(rd2, d3w, d3b), 0.0)
    xbar_o[...] = lin(rd3, xbw, xbb)
    r_o[...] = r
    re1_o[...] = re1
    q1_o[...] = _soft_assign_block(r, ct_ref[...])


def _ae_branch(x, weights, biases, cluster_t, bm=512):
    m, k = x.shape
    n_in = weights[7].shape[1]
    nz = weights[3].shape[1]
    e1 = weights[0].shape[1]
    nc = cluster_t.shape[1]

    def wspec(w):
        return pl.BlockSpec(w.shape, lambda i: (0, 0))

    operands = [x]
    in_specs = [pl.BlockSpec((bm, k), lambda i: (i, 0))]
    for w, b in zip(weights, biases):
        operands += [w, b]
        in_specs += [wspec(w), wspec(b)]
    operands.append(cluster_t)
    in_specs.append(wspec(cluster_t))

    out_shape = (
        jax.ShapeDtypeStruct((m, n_in), _F32),
        jax.ShapeDtypeStruct((m, nz), _F32),
        jax.ShapeDtypeStruct((m, e1), _F32),
        jax.ShapeDtypeStruct((m, nc), _F32),
    )
    out_specs = (
        pl.BlockSpec((bm, n_in), lambda i: (i, 0)),
        pl.BlockSpec((bm, nz), lambda i: (i, 0)),
        pl.BlockSpec((bm, e1), lambda i: (i, 0)),
        pl.BlockSpec((bm, nc), lambda i: (i, 0)),
    )
    return pl.pallas_call(
        _ae_body,
        grid=(m // bm,),
        in_specs=in_specs,
        out_specs=out_specs,
        out_shape=out_shape,
        compiler_params=pltpu.CompilerParams(
            dimension_semantics=("parallel",),
        ),
    )(*operands)


def _sa_body(h_ref, ct_ref, o_ref):
    o_ref[...] = _soft_assign_block(h_ref[...], ct_ref[...])


def _soft_assign(h, cluster_t):
    m, nz = h.shape
    nc = cluster_t.shape[1]
    return pl.pallas_call(
        _sa_body,
        out_shape=jax.ShapeDtypeStruct((m, nc), _F32),
    )(h, cluster_t)


def kernel(x, adj, enc1_w, enc1_b, enc2_w, enc2_b, enc3_w, enc3_b, zl_w, zl_b, dec1_w, dec1_b, dec2_w, dec2_b, dec3_w, dec3_b, xbar_w, xbar_b, g1_w, g2_w, g3_w, g4_w, g5_w, g6_w, g7_w, g8_w, g9_w, cluster):
    row = lambda b: b.reshape(1, -1)
    cluster_t = cluster.T

    x_bar, r, r_e1, q1 = _ae_branch(
        x,
        (enc1_w, enc2_w, enc3_w, zl_w, dec1_w, dec2_w, dec3_w, xbar_w),
        (row(enc1_b), row(enc2_b), row(enc3_b), row(zl_b), row(dec1_b),
         row(dec2_b), row(dec3_b), row(xbar_b)),
        cluster_t,
    )

    z1 = _mm(adj, _mm(x, g1_w), act="relu")
    h2 = _mm(z1 + r_e1, enc2_w, act="relu", bias=row(enc2_b))
    z2 = _mm(adj, _mm(z1, g2_w), act="relu")
    z3 = _mm(adj, _mm(z2 + h2, g3_w), act="relu")
    z = _mm(adj, _mm(z3, g4_w), act="relu")

    z_i = z + r
    c1 = _mm(z_i, g5_w)
    c3 = _mm(z, g6_w)
    pad = lambda t: jnp.pad(t, ((0, 0), (0, 128 - t.shape[1])))
    u5 = jnp.concatenate([pad(c1), pad(z_i), c3], axis=1)
    p5 = _mm(adj, u5, act="p5")
    a_r = p5[:, :10]
    z_l = p5[:, 128:138]
    dec_z1 = p5[:, 256:]

    dec_z2 = _mm(adj, _mm(dec_z1, g7_w), act="relu")
    dec_z3 = _mm(adj, _mm(dec_z2, g8_w), act="relu")
    z_hat = _mm(adj, _mm(dec_z3, g9_w), act="relu")

    adj_hat = _mm(z_hat, z_hat.T, act="sigmoid", bm=512)
    q = _soft_assign(z_l, cluster_t)

    return (x_bar, z_hat, adj_hat, q, q1, a_r, z, r, z_l)


# speedup vs baseline: 1.2104x; 1.2104x over previous
import functools

import jax
import jax.numpy as jnp
from jax.experimental import pallas as pl
from jax.experimental.pallas import tpu as pltpu

_F32 = jnp.float32
_BF16 = jnp.bfloat16


def _bf(t):
    return t.astype(_BF16)


def _soft_assign_block(h, c_t):
    d2 = (
        jnp.sum(h * h, axis=1, keepdims=True)
        - 2.0 * jnp.dot(h, c_t, preferred_element_type=_F32)
        + jnp.sum(c_t * c_t, axis=0, keepdims=True)
    )
    q = 1.0 / (1.0 + d2)
    return q / jnp.sum(q, axis=1, keepdims=True)


def _mm1_body(a_ref, b_ref, o_ref, *, act):
    acc = jnp.dot(_bf(a_ref[...]), _bf(b_ref[...]), preferred_element_type=_F32)
    o_ref[...] = _apply_act(acc, act).astype(o_ref.dtype)


def _mm1_bias_body(a_ref, b_ref, bias_ref, o_ref, *, act):
    acc = jnp.dot(_bf(a_ref[...]), _bf(b_ref[...]), preferred_element_type=_F32)
    o_ref[...] = _apply_act(acc + bias_ref[...], act).astype(o_ref.dtype)


def _mmk_body(a_ref, b_ref, o_ref, acc_ref, *, nk, act):
    k = pl.program_id(1)

    @pl.when(k == 0)
    def _init():
        acc_ref[...] = jnp.zeros_like(acc_ref)

    acc_ref[...] += jnp.dot(
        _bf(a_ref[...]), _bf(b_ref[...]), preferred_element_type=_F32
    )

    @pl.when(k == nk - 1)
    def _fin():
        o_ref[...] = _apply_act(acc_ref[...], act).astype(o_ref.dtype)


def _apply_act(acc, act):
    if act == "relu":
        return jnp.maximum(acc, 0.0)
    if act == "sigmoid":
        return jax.nn.sigmoid(acc)
    if act == "p5":
        lane = jax.lax.broadcasted_iota(jnp.int32, acc.shape, 1)
        keep = (lane >= 128) & (lane < 256)
        return jnp.where(keep, acc, jnp.maximum(acc, 0.0))
    return acc


def _mm(a, b, act="none", bm=1024, bk=512, bias=None, out_dtype=_F32):
    m, k = a.shape
    n = b.shape[1]
    bm = min(bm, m)
    if k <= 2048:
        grid = (m // bm,)
        in_specs = [
            pl.BlockSpec((bm, k), lambda i: (i, 0)),
            pl.BlockSpec((k, n), lambda i: (0, 0)),
        ]
        operands = [a, b]
        if bias is not None:
            body = functools.partial(_mm1_bias_body, act=act)
            in_specs.append(pl.BlockSpec((1, n), lambda i: (0, 0)))
            operands.append(bias)
        else:
            body = functools.partial(_mm1_body, act=act)
        return pl.pallas_call(
            body,
            grid=grid,
            in_specs=in_specs,
            out_specs=pl.BlockSpec((bm, n), lambda i: (i, 0)),
            out_shape=jax.ShapeDtypeStruct((m, n), out_dtype),
            compiler_params=pltpu.CompilerParams(
                dimension_semantics=("parallel",),
            ),
        )(*operands)
    nk = k // bk
    return pl.pallas_call(
        functools.partial(_mmk_body, nk=nk, act=act),
        grid=(m // bm, nk),
        in_specs=[
            pl.BlockSpec((bm, bk), lambda i, j: (i, j)),
            pl.BlockSpec((bk, n), lambda i, j: (j, 0)),
        ],
        out_specs=pl.BlockSpec((bm, n), lambda i, j: (i, 0)),
        out_shape=jax.ShapeDtypeStruct((m, n), out_dtype),
        scratch_shapes=[pltpu.VMEM((bm, n), _F32)],
        compiler_params=pltpu.CompilerParams(
            dimension_semantics=("parallel", "arbitrary"),
        ),
    )(a, b)


def _ae_body(
    x_ref,
    e1w, e1b, e2w, e2b, e3w, e3b, zlw, zlb,
    d1w, d1b, d2w, d2b, d3w, d3b, xbw, xbb,
    ct_ref,
    xbar_o, r_o, re1_o, q1_o,
):
    def lin(t, w, b):
        return jnp.dot(t, w[...], preferred_element_type=_F32) + b[...]

    x = x_ref[...]
    re1 = jnp.maximum(lin(x, e1w, e1b), 0.0)
    re2 = jnp.maximum(lin(re1, e2w, e2b), 0.0)
    re3 = jnp.maximum(lin(re2, e3w, e3b), 0.0)
    r = lin(re3, zlw, zlb)
    rd1 = jnp.maximum(lin(r, d1w, d1b), 0.0)
    rd2 = jnp.maximum(lin(rd1, d2w, d2b), 0.0)
    rd3 = jnp.maximum(lin(rd2, d3w, d3b), 0.0)
    xbar_o[...] = lin(rd3, xbw, xbb)
    r_o[...] = r
    re1_o[...] = re1
    q1_o[...] = _soft_assign_block(r, ct_ref[...])


def _ae_branch(x, weights, biases, cluster_t, bm=512):
    m, k = x.shape
    n_in = weights[7].shape[1]
    nz = weights[3].shape[1]
    e1 = weights[0].shape[1]
    nc = cluster_t.shape[1]

    def wspec(w):
        return pl.BlockSpec(w.shape, lambda i: (0, 0))

    operands = [x]
    in_specs = [pl.BlockSpec((bm, k), lambda i: (i, 0))]
    for w, b in zip(weights, biases):
        operands += [w, b]
        in_specs += [wspec(w), wspec(b)]
    operands.append(cluster_t)
    in_specs.append(wspec(cluster_t))

    out_shape = (
        jax.ShapeDtypeStruct((m, n_in), _F32),
        jax.ShapeDtypeStruct((m, nz), _F32),
        jax.ShapeDtypeStruct((m, e1), _F32),
        jax.ShapeDtypeStruct((m, nc), _F32),
    )
    out_specs = (
        pl.BlockSpec((bm, n_in), lambda i: (i, 0)),
        pl.BlockSpec((bm, nz), lambda i: (i, 0)),
        pl.BlockSpec((bm, e1), lambda i: (i, 0)),
        pl.BlockSpec((bm, nc), lambda i: (i, 0)),
    )
    return pl.pallas_call(
        _ae_body,
        grid=(m // bm,),
        in_specs=in_specs,
        out_specs=out_specs,
        out_shape=out_shape,
        compiler_params=pltpu.CompilerParams(
            dimension_semantics=("parallel",),
        ),
    )(*operands)


def _sa_body(h_ref, ct_ref, o_ref):
    o_ref[...] = _soft_assign_block(h_ref[...], ct_ref[...])


def _soft_assign(h, cluster_t):
    m, nz = h.shape
    nc = cluster_t.shape[1]
    return pl.pallas_call(
        _sa_body,
        out_shape=jax.ShapeDtypeStruct((m, nc), _F32),
    )(h, cluster_t)


def kernel(x, adj, enc1_w, enc1_b, enc2_w, enc2_b, enc3_w, enc3_b, zl_w, zl_b, dec1_w, dec1_b, dec2_w, dec2_b, dec3_w, dec3_b, xbar_w, xbar_b, g1_w, g2_w, g3_w, g4_w, g5_w, g6_w, g7_w, g8_w, g9_w, cluster):
    row = lambda b: b.reshape(1, -1)
    cluster_t = cluster.T
    adj_b = _bf(adj)

    x_bar, r, r_e1, q1 = _ae_branch(
        x,
        (enc1_w, enc2_w, enc3_w, zl_w, dec1_w, dec2_w, dec3_w, xbar_w),
        (row(enc1_b), row(enc2_b), row(enc3_b), row(zl_b), row(dec1_b),
         row(dec2_b), row(dec3_b), row(xbar_b)),
        cluster_t,
    )

    bfo = dict(out_dtype=_BF16)
    z1 = _mm(adj_b, _mm(x, _bf(g1_w), **bfo), act="relu", bm=2048, **bfo)
    h2 = _mm(z1 + r_e1, _bf(enc2_w), act="relu", bias=row(enc2_b), **bfo)
    z2 = _mm(adj_b, _mm(z1, _bf(g2_w), **bfo), act="relu", bm=2048, **bfo)
    z3 = _mm(adj_b, _mm(z2 + h2, _bf(g3_w), **bfo), act="relu", **bfo)
    z = _mm(adj_b, _mm(z3, _bf(g4_w), **bfo), act="relu", bm=2048)

    z_i = z + r
    c1 = _mm(z_i, _bf(g5_w), **bfo)
    c3 = _mm(z, _bf(g6_w), **bfo)
    pad = lambda t: jnp.pad(t, ((0, 0), (0, 128 - t.shape[1])))
    u5 = jnp.concatenate([pad(c1), pad(_bf(z_i)), c3], axis=1)
    p5 = _mm(adj_b, u5, act="p5")
    a_r = p5[:, :10]
    z_l = p5[:, 128:138]
    dec_z1 = p5[:, 256:]

    dec_z2 = _mm(adj_b, _mm(_bf(dec_z1), _bf(g7_w), **bfo), act="relu", bm=2048, **bfo)
    dec_z3 = _mm(adj_b, _mm(dec_z2, _bf(g8_w), **bfo), act="relu", bm=2048, **bfo)
    z_hat = _mm(adj_b, _mm(dec_z3, _bf(g9_w), **bfo), act="relu", bm=2048)

    zh_b = _bf(z_hat)
    adj_hat = _mm(zh_b, zh_b.T, act="sigmoid", bm=1024)
    q = _soft_assign(z_l, cluster_t)

    return (x_bar, z_hat, adj_hat, q, q1, a_r, z, r, z_l)


# speedup vs baseline: 1.4512x; 1.1990x over previous
import functools

import jax
import jax.numpy as jnp
from jax.experimental import pallas as pl
from jax.experimental.pallas import tpu as pltpu

_F32 = jnp.float32
_BF16 = jnp.bfloat16
_N = 4096
_BK = 512
_NK = _N // _BK


def _bf(t):
    return t.astype(_BF16)


def _soft_assign_block(h, c_t):
    d2 = (
        jnp.sum(h * h, axis=1, keepdims=True)
        - 2.0 * jnp.dot(h, c_t, preferred_element_type=_F32)
        + jnp.sum(c_t * c_t, axis=0, keepdims=True)
    )
    q = 1.0 / (1.0 + d2)
    return q / jnp.sum(q, axis=1, keepdims=True)


def _conv_body(*refs, nk, act, n_a, has_w, emit_adj, mode):
    it = iter(refs)
    adj_ref = next(it)
    a_refs = [next(it) for _ in range(n_a)]
    w_ref = next(it) if has_w else None
    ct_ref = next(it) if mode == "p5" else None
    if mode == "p5":
        outs = [next(it) for _ in range(4)]
    elif mode == "dual":
        outs = [next(it), next(it)]
    else:
        outs = [next(it)]
    adjbf_out = next(it) if emit_adj else None
    acc_ref = next(it)

    k = pl.program_id(1)

    @pl.when(k == 0)
    def _init():
        acc_ref[...] = jnp.zeros_like(acc_ref)

    a_bf = _bf(adj_ref[...])
    if emit_adj:
        adjbf_out[...] = a_bf

    if has_w:
        xin = a_refs[0][...]
        for extra in a_refs[1:]:
            xin = xin + extra[...]
        u = _bf(jnp.dot(_bf(xin), w_ref[...], preferred_element_type=_F32))
    else:
        u = a_refs[0][...]

    acc_ref[...] += jnp.dot(a_bf, u, preferred_element_type=_F32)

    @pl.when(k == nk - 1)
    def _fin():
        acc = acc_ref[...]
        if mode == "p5":
            outs[0][...] = jnp.maximum(acc[:, 0:10], 0.0)
            zl = acc[:, 128:138]
            outs[1][...] = zl
            outs[2][...] = _soft_assign_block(zl, ct_ref[...])
            outs[3][...] = _bf(jnp.maximum(acc[:, 256:2256], 0.0))
        else:
            if act == "relu":
                acc = jnp.maximum(acc, 0.0)
            outs[0][...] = acc.astype(outs[0].dtype)
            if mode == "dual":
                outs[1][...] = acc.astype(outs[1].dtype)


def _conv(adj, a_list, w, *, act="relu", bm, out_n, out_dtype=_BF16,
          emit_adj=False, mode="plain", cluster_t=None):
    m = adj.shape[0]
    nm = m // bm
    has_w = w is not None
    n_a = len(a_list)
    d_in = a_list[0].shape[1]

    in_specs = [pl.BlockSpec((bm, _BK), lambda i, j: (i, j))]
    operands = [adj]
    for a in a_list:
        in_specs.append(pl.BlockSpec((_BK, d_in), lambda i, j: (j, 0)))
        operands.append(a)
    if has_w:
        in_specs.append(pl.BlockSpec(w.shape, lambda i, j: (0, 0)))
        operands.append(w)
    if mode == "p5":
        in_specs.append(pl.BlockSpec(cluster_t.shape, lambda i, j: (0, 0)))
        operands.append(cluster_t)

    def ospec(n):
        return pl.BlockSpec((bm, n), lambda i, j: (i, 0))

    if mode == "p5":
        out_shape = (
            jax.ShapeDtypeStruct((m, 10), _F32),
            jax.ShapeDtypeStruct((m, 10), _F32),
            jax.ShapeDtypeStruct((m, 10), _F32),
            jax.ShapeDtypeStruct((m, 2000), _BF16),
        )
        out_specs = (ospec(10), ospec(10), ospec(10), ospec(2000))
    elif mode == "dual":
        out_shape = (
            jax.ShapeDtypeStruct((m, out_n), _F32),
            jax.ShapeDtypeStruct((m, out_n), _BF16),
        )
        out_specs = (ospec(out_n), ospec(out_n))
    else:
        out_shape = (jax.ShapeDtypeStruct((m, out_n), out_dtype),)
        out_specs = (ospec(out_n),)
    if emit_adj:
        out_shape = out_shape + (jax.ShapeDtypeStruct((m, m), _BF16),)
        out_specs = out_specs + (pl.BlockSpec((bm, _BK), lambda i, j: (i, j)),)

    acc_n = 2256 if mode == "p5" else out_n
    body = functools.partial(
        _conv_body, nk=_NK, act=act, n_a=n_a, has_w=has_w,
        emit_adj=emit_adj, mode=mode,
    )
    res = pl.pallas_call(
        body,
        grid=(nm, _NK),
        in_specs=in_specs,
        out_specs=out_specs,
        out_shape=out_shape,
        scratch_shapes=[pltpu.VMEM((bm, acc_n), _F32)],
        compiler_params=pltpu.CompilerParams(
            dimension_semantics=("parallel", "arbitrary"),
        ),
    )(*operands)
    return res


def _ae_body(
    x_ref,
    e1w, e1b, e2w, e2b, e3w, e3b, zlw, zlb,
    d1w, d1b, d2w, d2b, d3w, d3b, xbw, xbb,
    g1w, ct_ref,
    xbar_o, r_o, re1_o, q1_o, u1_o,
):
    def lin(t, w, b):
        return jnp.dot(_bf(t), w[...], preferred_element_type=_F32) + b[...]

    x = x_ref[...]
    re1 = jnp.maximum(lin(x, e1w, e1b), 0.0)
    re2 = jnp.maximum(lin(re1, e2w, e2b), 0.0)
    re3 = jnp.maximum(lin(re2, e3w, e3b), 0.0)
    r = lin(re3, zlw, zlb)
    rd1 = jnp.maximum(lin(r, d1w, d1b), 0.0)
    rd2 = jnp.maximum(lin(rd1, d2w, d2b), 0.0)
    rd3 = jnp.maximum(lin(rd2, d3w, d3b), 0.0)
    xbar_o[...] = lin(rd3, xbw, xbb)
    r_o[...] = r
    re1_o[...] = _bf(re1)
    q1_o[...] = _soft_assign_block(r, ct_ref[...])
    u1_o[...] = _bf(jnp.dot(_bf(x), g1w[...], preferred_element_type=_F32))


def _ae_branch(x, weights, biases, g1_w, cluster_t, bm=512):
    m, k = x.shape
    n_in = weights[7].shape[1]
    nz = weights[3].shape[1]
    e1 = weights[0].shape[1]
    nc = cluster_t.shape[1]

    def wspec(w):
        return pl.BlockSpec(w.shape, lambda i: (0, 0))

    operands = [x]
    in_specs = [pl.BlockSpec((bm, k), lambda i: (i, 0))]
    for w, b in zip(weights, biases):
        operands += [w, b]
        in_specs += [wspec(w), wspec(b)]
    operands += [g1_w, cluster_t]
    in_specs += [wspec(g1_w), wspec(cluster_t)]

    def ospec(n):
        return pl.BlockSpec((bm, n), lambda i: (i, 0))

    out_shape = (
        jax.ShapeDtypeStruct((m, n_in), _F32),
        jax.ShapeDtypeStruct((m, nz), _F32),
        jax.ShapeDtypeStruct((m, e1), _BF16),
        jax.ShapeDtypeStruct((m, nc), _F32),
        jax.ShapeDtypeStruct((m, e1), _BF16),
    )
    out_specs = (ospec(n_in), ospec(nz), ospec(e1), ospec(nc), ospec(e1))
    return pl.pallas_call(
        _ae_body,
        grid=(m // bm,),
        in_specs=in_specs,
        out_specs=out_specs,
        out_shape=out_shape,
        compiler_params=pltpu.CompilerParams(
            dimension_semantics=("parallel",),
        ),
    )(*operands)


def _h2_body(a1_ref, a2_ref, w_ref, b_ref, o_ref):
    s = a1_ref[...] + a2_ref[...]
    acc = jnp.dot(s, w_ref[...], preferred_element_type=_F32) + b_ref[...]
    o_ref[...] = _bf(jnp.maximum(acc, 0.0))


def _h2(z1, r_e1, w, b, bm=2048):
    m, k = z1.shape
    n = w.shape[1]
    return pl.pallas_call(
        _h2_body,
        grid=(m // bm,),
        in_specs=[
            pl.BlockSpec((bm, k), lambda i: (i, 0)),
            pl.BlockSpec((bm, k), lambda i: (i, 0)),
            pl.BlockSpec(w.shape, lambda i: (0, 0)),
            pl.BlockSpec(b.shape, lambda i: (0, 0)),
        ],
        out_specs=pl.BlockSpec((bm, n), lambda i: (i, 0)),
        out_shape=jax.ShapeDtypeStruct((m, n), _BF16),
        compiler_params=pltpu.CompilerParams(
            dimension_semantics=("parallel",),
        ),
    )(z1, r_e1, w, b)


def _nt_body(a_ref, b_ref, o_ref):
    o = jax.lax.dot_general(
        a_ref[...], b_ref[...], (((1,), (1,)), ((), ())),
        preferred_element_type=_F32,
    )
    o_ref[...] = jax.nn.sigmoid(o)


def _adj_hat(zh_bf, bm=1024):
    m, k = zh_bf.shape
    return pl.pallas_call(
        _nt_body,
        grid=(m // bm,),
        in_specs=[
            pl.BlockSpec((bm, k), lambda i: (i, 0)),
            pl.BlockSpec((m, k), lambda i: (0, 0)),
        ],
        out_specs=pl.BlockSpec((bm, m), lambda i: (i, 0)),
        out_shape=jax.ShapeDtypeStruct((m, m), _F32),
        compiler_params=pltpu.CompilerParams(
            dimension_semantics=("parallel",),
        ),
    )(zh_bf, zh_bf)


def kernel(x, adj, enc1_w, enc1_b, enc2_w, enc2_b, enc3_w, enc3_b, zl_w, zl_b, dec1_w, dec1_b, dec2_w, dec2_b, dec3_w, dec3_b, xbar_w, xbar_b, g1_w, g2_w, g3_w, g4_w, g5_w, g6_w, g7_w, g8_w, g9_w, cluster):
    row = lambda b: b.reshape(1, -1)
    cluster_t = cluster.T
    nz = cluster.shape[1]

    x_bar, r, r_e1, q1, u1 = _ae_branch(
        x,
        (_bf(enc1_w), _bf(enc2_w), _bf(enc3_w), _bf(zl_w), _bf(dec1_w),
         _bf(dec2_w), _bf(dec3_w), _bf(xbar_w)),
        (row(enc1_b), row(enc2_b), row(enc3_b), row(zl_b), row(dec1_b),
         row(dec2_b), row(dec3_b), row(xbar_b)),
        _bf(g1_w), cluster_t,
    )

    z1, adj_b = _conv(adj, [u1], None, bm=4096, out_n=500, emit_adj=True)
    h2 = _h2(z1, r_e1, _bf(enc2_w), row(enc2_b))
    z2 = _conv(adj_b, [z1], _bf(g2_w), bm=4096, out_n=500)[0]
    z3 = _conv(adj_b, [z2, h2], _bf(g3_w), bm=2048, out_n=2000)[0]
    z = _conv(adj_b, [z3], _bf(g4_w), bm=4096, out_n=nz, out_dtype=_F32)[0]

    z_i = z + r
    azi = _bf(jnp.concatenate([z_i, z], axis=1))
    w5 = jnp.zeros((2 * nz, 2256), _F32)
    w5 = w5.at[:nz, :nz].set(g5_w)
    w5 = w5.at[:nz, 128:128 + nz].set(jnp.eye(nz, dtype=_F32))
    w5 = w5.at[nz:, 256:].set(g6_w)
    a_r, z_l, q, dec_z1 = _conv(
        adj_b, [azi], _bf(w5), bm=1024, out_n=2256, mode="p5",
        cluster_t=cluster_t,
    )[:4]

    dec_z2 = _conv(adj_b, [dec_z1], _bf(g7_w), bm=4096, out_n=500)[0]
    dec_z3 = _conv(adj_b, [dec_z2], _bf(g8_w), bm=4096, out_n=500)[0]
    z_hat, zh_bf = _conv(adj_b, [dec_z3], _bf(g9_w), bm=4096, out_n=512,
                         mode="dual")[:2]

    adj_hat = _adj_hat(zh_bf)

    return (x_bar, z_hat, adj_hat, q, q1, a_r, z, r, z_l)


# speedup vs baseline: 1.8985x; 1.3083x over previous
import jax
import jax.numpy as jnp
from jax.experimental import pallas as pl
from jax.experimental.pallas import tpu as pltpu

_F32 = jnp.float32
_BF16 = jnp.bfloat16
_N = 4096
_BK = 512
_NK = _N // _BK


def _bf(t):
    return t.astype(_BF16)


def _dot(a, b):
    return jnp.dot(_bf(a), _bf(b), preferred_element_type=_F32)


def _soft_assign_block(h, c_t):
    d2 = (
        jnp.sum(h * h, axis=1, keepdims=True)
        - 2.0 * jnp.dot(h, c_t, preferred_element_type=_F32)
        + jnp.sum(c_t * c_t, axis=0, keepdims=True)
    )
    q = 1.0 / (1.0 + d2)
    return q / jnp.sum(q, axis=1, keepdims=True)


def _sweep(adj, u, extras, out_defs, epilogue, *, bm=2048, emit_adj=False):
    m = adj.shape[0]
    nm = m // bm
    du = u.shape[1]
    n_extra = len(extras)

    def body(*refs):
        it = iter(refs)
        adj_ref = next(it)
        u_ref = next(it)
        extra_refs = [next(it) for _ in range(n_extra)]
        outs = [next(it) for _ in range(len(out_defs))]
        adjbf_out = next(it) if emit_adj else None
        acc_ref = next(it)

        k = pl.program_id(1)

        @pl.when(k == 0)
        def _init():
            acc_ref[...] = jnp.zeros_like(acc_ref)

        a_blk = _bf(adj_ref[...])
        if emit_adj:
            adjbf_out[...] = a_blk
        acc_ref[...] += jnp.dot(a_blk, u_ref[...], preferred_element_type=_F32)

        @pl.when(k == _NK - 1)
        def _fin():
            results = epilogue(acc_ref[...], [e[...] for e in extra_refs])
            for o_ref, val in zip(outs, results):
                o_ref[...] = val.astype(o_ref.dtype)

    in_specs = [
        pl.BlockSpec((bm, _BK), lambda i, j: (i, j)),
        pl.BlockSpec((_BK, du), lambda i, j: (j, 0)),
    ]
    operands = [adj, u]
    for arr, kind in extras:
        if kind == "row":
            in_specs.append(pl.BlockSpec((bm, arr.shape[1]), lambda i, j: (i, 0)))
        else:
            in_specs.append(pl.BlockSpec(arr.shape, lambda i, j: (0, 0)))
        operands.append(arr)

    out_shape = tuple(jax.ShapeDtypeStruct((m, n), dt) for n, dt in out_defs)
    out_specs = tuple(
        pl.BlockSpec((bm, n), lambda i, j: (i, 0)) for n, _ in out_defs
    )
    if emit_adj:
        out_shape = out_shape + (jax.ShapeDtypeStruct((m, m), _BF16),)
        out_specs = out_specs + (pl.BlockSpec((bm, _BK), lambda i, j: (i, j)),)

    return pl.pallas_call(
        body,
        grid=(nm, _NK),
        in_specs=in_specs,
        out_specs=out_specs,
        out_shape=out_shape,
        scratch_shapes=[pltpu.VMEM((bm, du), _F32)],
        compiler_params=pltpu.CompilerParams(
            dimension_semantics=("parallel", "arbitrary"),
        ),
    )(*operands)


def _ae_body(
    x_ref,
    e1w, e1b, e2w, e2b, e3w, e3b, zlw, zlb,
    d1w, d1b, d2w, d2b, d3w, d3b, xbw, xbb,
    g1w, ct_ref,
    xbar_o, r_o, re1_o, q1_o, u1_o,
):
    def lin(t, w, b):
        return _dot(t, w[...]) + b[...]

    x = x_ref[...]
    re1 = jnp.maximum(lin(x, e1w, e1b), 0.0)
    re2 = jnp.maximum(lin(re1, e2w, e2b), 0.0)
    re3 = jnp.maximum(lin(re2, e3w, e3b), 0.0)
    r = lin(re3, zlw, zlb)
    rd1 = jnp.maximum(lin(r, d1w, d1b), 0.0)
    rd2 = jnp.maximum(lin(rd1, d2w, d2b), 0.0)
    rd3 = jnp.maximum(lin(rd2, d3w, d3b), 0.0)
    xbar_o[...] = lin(rd3, xbw, xbb)
    r_o[...] = r
    re1_o[...] = _bf(re1)
    q1_o[...] = _soft_assign_block(r, ct_ref[...])
    u1_o[...] = _bf(_dot(x, g1w[...]))


def _ae_branch(x, weights, biases, g1_w, cluster_t, bm=512):
    m, k = x.shape
    n_in = weights[7].shape[1]
    nz = weights[3].shape[1]
    e1 = weights[0].shape[1]
    nc = cluster_t.shape[1]

    def wspec(w):
        return pl.BlockSpec(w.shape, lambda i: (0, 0))

    operands = [x]
    in_specs = [pl.BlockSpec((bm, k), lambda i: (i, 0))]
    for w, b in zip(weights, biases):
        operands += [w, b]
        in_specs += [wspec(w), wspec(b)]
    operands += [g1_w, cluster_t]
    in_specs += [wspec(g1_w), wspec(cluster_t)]

    def ospec(n):
        return pl.BlockSpec((bm, n), lambda i: (i, 0))

    out_shape = (
        jax.ShapeDtypeStruct((m, n_in), _F32),
        jax.ShapeDtypeStruct((m, nz), _F32),
        jax.ShapeDtypeStruct((m, e1), _BF16),
        jax.ShapeDtypeStruct((m, nc), _F32),
        jax.ShapeDtypeStruct((m, e1), _BF16),
    )
    out_specs = (ospec(n_in), ospec(nz), ospec(e1), ospec(nc), ospec(e1))
    return pl.pallas_call(
        _ae_body,
        grid=(m // bm,),
        in_specs=in_specs,
        out_specs=out_specs,
        out_shape=out_shape,
        compiler_params=pltpu.CompilerParams(
            dimension_semantics=("parallel",),
        ),
    )(*operands)


def _nt_body(a_ref, b_ref, o_ref):
    o = jax.lax.dot_general(
        a_ref[...], b_ref[...], (((1,), (1,)), ((), ())),
        preferred_element_type=_F32,
    )
    o_ref[...] = jax.nn.sigmoid(o)


def _adj_hat(zh_bf, bm=1024):
    m, k = zh_bf.shape
    return pl.pallas_call(
        _nt_body,
        grid=(m // bm,),
        in_specs=[
            pl.BlockSpec((bm, k), lambda i: (i, 0)),
            pl.BlockSpec((m, k), lambda i: (0, 0)),
        ],
        out_specs=pl.BlockSpec((bm, m), lambda i: (i, 0)),
        out_shape=jax.ShapeDtypeStruct((m, m), _F32),
        compiler_params=pltpu.CompilerParams(
            dimension_semantics=("parallel",),
        ),
    )(zh_bf, zh_bf)


def kernel(x, adj, enc1_w, enc1_b, enc2_w, enc2_b, enc3_w, enc3_b, zl_w, zl_b, dec1_w, dec1_b, dec2_w, dec2_b, dec3_w, dec3_b, xbar_w, xbar_b, g1_w, g2_w, g3_w, g4_w, g5_w, g6_w, g7_w, g8_w, g9_w, cluster):
    row = lambda b: b.reshape(1, -1)
    cluster_t = cluster.T
    nz = cluster.shape[1]
    relu = lambda t: jnp.maximum(t, 0.0)

    x_bar, r, r_e1, q1, u1 = _ae_branch(
        x,
        (_bf(enc1_w), _bf(enc2_w), _bf(enc3_w), _bf(zl_w), _bf(dec1_w),
         _bf(dec2_w), _bf(dec3_w), _bf(xbar_w)),
        (row(enc1_b), row(enc2_b), row(enc3_b), row(zl_b), row(dec1_b),
         row(dec2_b), row(dec3_b), row(xbar_b)),
        _bf(g1_w), cluster_t,
    )

    def ep1(acc, ex):
        re1, e2w, e2b, g2w = ex
        z1 = relu(acc)
        h2 = relu(_dot(z1 + re1.astype(_F32), e2w) + e2b)
        return [h2, _dot(z1, g2w)]

    h2, u2, adj_b = _sweep(
        adj, u1,
        [(r_e1, "row"), (_bf(enc2_w), "full"), (row(enc2_b), "full"),
         (_bf(g2_w), "full")],
        [(500, _BF16), (500, _BF16)], ep1, emit_adj=True,
    )

    def ep2(acc, ex):
        (h2v,) = ex
        return [relu(acc) + h2v.astype(_F32)]

    (a3,) = _sweep(adj_b, u2, [(h2, "row")], [(500, _BF16)], ep2)

    def ep3(acc, ex):
        g3w, g4w = ex
        z3 = relu(_dot(acc, g3w))
        return [_dot(z3, g4w)]

    (u4,) = _sweep(adj_b, a3, [(_bf(g3_w), "full"), (_bf(g4_w), "full")],
                   [(nz, _BF16)], ep3)

    def ep4(acc, ex):
        return [relu(acc)]

    (z,) = _sweep(adj_b, u4, [], [(nz, _F32)], ep4)

    pad = lambda t: jnp.pad(t, ((0, 0), (0, 128 - t.shape[1])))
    azr = _bf(jnp.concatenate([pad(z), pad(r)], axis=1))

    def ep5(acc, ex):
        g5w, g6w, g7w, ct = ex
        t1 = acc[:, 0:nz]
        zl = t1 + acc[:, 128:128 + nz]
        a_r = relu(_dot(zl, g5w))
        q = _soft_assign_block(zl, ct)
        dec_z1 = relu(_dot(t1, g6w))
        return [a_r, zl, q, _dot(dec_z1, g7w)]

    a_r, z_l, q, u7 = _sweep(
        adj_b, azr,
        [(_bf(g5_w), "full"), (_bf(g6_w), "full"), (_bf(g7_w), "full"),
         (cluster_t, "full")],
        [(nz, _F32), (nz, _F32), (nz, _F32), (500, _BF16)], ep5, bm=1024,
    )

    def ep6(acc, ex):
        (g8w,) = ex
        return [_dot(relu(acc), g8w)]

    (u8,) = _sweep(adj_b, u7, [(_bf(g8_w), "full")], [(500, _BF16)], ep6)

    def ep7(acc, ex):
        (g9w,) = ex
        return [_dot(relu(acc), g9w)]

    (u9,) = _sweep(adj_b, u8, [(_bf(g9_w), "full")], [(512, _BF16)], ep7)

    def ep8(acc, ex):
        zh = relu(acc)
        return [zh, zh]

    z_hat, zh_bf = _sweep(adj_b, u9, [], [(512, _F32), (512, _BF16)], ep8)

    adj_hat = _adj_hat(zh_bf)

    return (x_bar, z_hat, adj_hat, q, q1, a_r, z, r, z_l)


# speedup vs baseline: 2.1270x; 1.1203x over previous
import jax
import jax.numpy as jnp
from jax.experimental import pallas as pl
from jax.experimental.pallas import tpu as pltpu

_F32 = jnp.float32
_BF16 = jnp.bfloat16
_N = 4096
_BK = 512
_NK = _N // _BK


def _bf(t):
    return t.astype(_BF16)


def _dot(a, b):
    return jnp.dot(_bf(a), _bf(b), preferred_element_type=_F32)


def _soft_assign_block(h, c_t):
    d2 = (
        jnp.sum(h * h, axis=1, keepdims=True)
        - 2.0 * jnp.dot(h, c_t, preferred_element_type=_F32)
        + jnp.sum(c_t * c_t, axis=0, keepdims=True)
    )
    q = 1.0 / (1.0 + d2)
    return q / jnp.sum(q, axis=1, keepdims=True)


def _sweep(adj, u, extras, out_defs, epilogue, *, bm=1024, emit_adj=False):
    m = adj.shape[0]
    nm = m // bm
    du = u.shape[1]
    n_extra = len(extras)

    def body(*refs):
        it = iter(refs)
        adj_ref = next(it)
        u_ref = next(it)
        extra_refs = [next(it) for _ in range(n_extra)]
        outs = [next(it) for _ in range(len(out_defs))]
        adjbf_out = next(it) if emit_adj else None

        a_blk = _bf(adj_ref[...])
        if emit_adj:
            adjbf_out[...] = a_blk
        s = jnp.dot(a_blk, u_ref[...], preferred_element_type=_F32)
        results = epilogue(s, [e[...] for e in extra_refs])
        for o_ref, val in zip(outs, results):
            o_ref[...] = val.astype(o_ref.dtype)

    in_specs = [
        pl.BlockSpec((bm, m), lambda i: (i, 0)),
        pl.BlockSpec((m, du), lambda i: (0, 0)),
    ]
    operands = [adj, u]
    for arr, kind in extras:
        if kind == "row":
            in_specs.append(pl.BlockSpec((bm, arr.shape[1]), lambda i: (i, 0)))
        else:
            in_specs.append(pl.BlockSpec(arr.shape, lambda i: (0, 0)))
        operands.append(arr)

    out_shape = tuple(jax.ShapeDtypeStruct((m, n), dt) for n, dt in out_defs)
    out_specs = tuple(
        pl.BlockSpec((bm, n), lambda i: (i, 0)) for n, _ in out_defs
    )
    if emit_adj:
        out_shape = out_shape + (jax.ShapeDtypeStruct((m, m), _BF16),)
        out_specs = out_specs + (pl.BlockSpec((bm, m), lambda i: (i, 0)),)

    return pl.pallas_call(
        body,
        grid=(nm,),
        in_specs=in_specs,
        out_specs=out_specs,
        out_shape=out_shape,
        compiler_params=pltpu.CompilerParams(
            dimension_semantics=("parallel",),
        ),
    )(*operands)


def _ae_body(
    x_ref,
    e1w, e1b, e2w, e2b, e3w, e3b, zlw, zlb,
    d1w, d1b, d2w, d2b, d3w, d3b, xbw, xbb,
    g1w, ct_ref,
    xbar_o, r_o, re1_o, q1_o, u1_o,
):
    def lin(t, w, b):
        return _dot(t, w[...]) + b[...]

    x = x_ref[...]
    re1 = jnp.maximum(lin(x, e1w, e1b), 0.0)
    re2 = jnp.maximum(lin(re1, e2w, e2b), 0.0)
    re3 = jnp.maximum(lin(re2, e3w, e3b), 0.0)
    r = lin(re3, zlw, zlb)
    rd1 = jnp.maximum(lin(r, d1w, d1b), 0.0)
    rd2 = jnp.maximum(lin(rd1, d2w, d2b), 0.0)
    rd3 = jnp.maximum(lin(rd2, d3w, d3b), 0.0)
    xbar_o[...] = lin(rd3, xbw, xbb)
    r_o[...] = r
    re1_o[...] = _bf(re1)
    q1_o[...] = _soft_assign_block(r, ct_ref[...])
    u1_o[...] = _bf(_dot(x, g1w[...]))


def _ae_branch(x, weights, biases, g1_w, cluster_t, bm=1024):
    m, k = x.shape
    n_in = weights[7].shape[1]
    nz = weights[3].shape[1]
    e1 = weights[0].shape[1]
    nc = cluster_t.shape[1]

    def wspec(w):
        return pl.BlockSpec(w.shape, lambda i: (0, 0))

    operands = [x]
    in_specs = [pl.BlockSpec((bm, k), lambda i: (i, 0))]
    for w, b in zip(weights, biases):
        operands += [w, b]
        in_specs += [wspec(w), wspec(b)]
    operands += [g1_w, cluster_t]
    in_specs += [wspec(g1_w), wspec(cluster_t)]

    def ospec(n):
        return pl.BlockSpec((bm, n), lambda i: (i, 0))

    out_shape = (
        jax.ShapeDtypeStruct((m, n_in), _F32),
        jax.ShapeDtypeStruct((m, nz), _F32),
        jax.ShapeDtypeStruct((m, e1), _BF16),
        jax.ShapeDtypeStruct((m, nc), _F32),
        jax.ShapeDtypeStruct((m, e1), _BF16),
    )
    out_specs = (ospec(n_in), ospec(nz), ospec(e1), ospec(nc), ospec(e1))
    return pl.pallas_call(
        _ae_body,
        grid=(m // bm,),
        in_specs=in_specs,
        out_specs=out_specs,
        out_shape=out_shape,
        compiler_params=pltpu.CompilerParams(
            dimension_semantics=("parallel",),
        ),
    )(*operands)


def _nt_body(a_ref, b_ref, o_ref):
    o = jax.lax.dot_general(
        a_ref[...], b_ref[...], (((1,), (1,)), ((), ())),
        preferred_element_type=_F32,
    )
    o_ref[...] = 0.5 * (1.0 + jnp.tanh(0.5 * o))


def _adj_hat(zh_bf, bm=1024):
    m, k = zh_bf.shape
    return pl.pallas_call(
        _nt_body,
        grid=(m // bm,),
        in_specs=[
            pl.BlockSpec((bm, k), lambda i: (i, 0)),
            pl.BlockSpec((m, k), lambda i: (0, 0)),
        ],
        out_specs=pl.BlockSpec((bm, m), lambda i: (i, 0)),
        out_shape=jax.ShapeDtypeStruct((m, m), _F32),
        compiler_params=pltpu.CompilerParams(
            dimension_semantics=("parallel",),
        ),
    )(zh_bf, zh_bf)


def kernel(x, adj, enc1_w, enc1_b, enc2_w, enc2_b, enc3_w, enc3_b, zl_w, zl_b, dec1_w, dec1_b, dec2_w, dec2_b, dec3_w, dec3_b, xbar_w, xbar_b, g1_w, g2_w, g3_w, g4_w, g5_w, g6_w, g7_w, g8_w, g9_w, cluster):
    row = lambda b: b.reshape(1, -1)
    cluster_t = cluster.T
    nz = cluster.shape[1]
    relu = lambda t: jnp.maximum(t, 0.0)

    x_bar, r, r_e1, q1, u1 = _ae_branch(
        x,
        (_bf(enc1_w), _bf(enc2_w), _bf(enc3_w), _bf(zl_w), _bf(dec1_w),
         _bf(dec2_w), _bf(dec3_w), _bf(xbar_w)),
        (row(enc1_b), row(enc2_b), row(enc3_b), row(zl_b), row(dec1_b),
         row(dec2_b), row(dec3_b), row(xbar_b)),
        _bf(g1_w), cluster_t,
    )

    def ep1(acc, ex):
        re1, e2w, e2b, g2w = ex
        z1 = relu(acc)
        h2 = relu(_dot(z1 + re1.astype(_F32), e2w) + e2b)
        return [h2, _dot(z1, g2w)]

    h2, u2, adj_b = _sweep(
        adj, u1,
        [(r_e1, "row"), (_bf(enc2_w), "full"), (row(enc2_b), "full"),
         (_bf(g2_w), "full")],
        [(500, _BF16), (500, _BF16)], ep1, bm=512, emit_adj=True,
    )

    def ep2(acc, ex):
        (h2v,) = ex
        return [relu(acc) + h2v.astype(_F32)]

    (a3,) = _sweep(adj_b, u2, [(h2, "row")], [(500, _BF16)], ep2)

    def ep3(acc, ex):
        g3w, g4w = ex
        z3 = relu(_dot(acc, g3w))
        return [_dot(z3, g4w)]

    (u4,) = _sweep(adj_b, a3, [(_bf(g3_w), "full"), (_bf(g4_w), "full")],
                   [(nz, _BF16)], ep3)

    def ep4(acc, ex):
        return [relu(acc)]

    (z,) = _sweep(adj_b, u4, [], [(nz, _F32)], ep4)

    pad = lambda t: jnp.pad(t, ((0, 0), (0, 128 - t.shape[1])))
    azr = _bf(jnp.concatenate([pad(z), pad(r)], axis=1))

    def ep5(acc, ex):
        g5w, g6w, g7w, ct = ex
        t1 = acc[:, 0:nz]
        zl = t1 + acc[:, 128:128 + nz]
        a_r = relu(_dot(zl, g5w))
        q = _soft_assign_block(zl, ct)
        dec_z1 = relu(_dot(t1, g6w))
        return [a_r, zl, q, _dot(dec_z1, g7w)]

    a_r, z_l, q, u7 = _sweep(
        adj_b, azr,
        [(_bf(g5_w), "full"), (_bf(g6_w), "full"), (_bf(g7_w), "full"),
         (cluster_t, "full")],
        [(nz, _F32), (nz, _F32), (nz, _F32), (500, _BF16)], ep5,
    )

    def ep6(acc, ex):
        (g8w,) = ex
        return [_dot(relu(acc), g8w)]

    (u8,) = _sweep(adj_b, u7, [(_bf(g8_w), "full")], [(500, _BF16)], ep6)

    def ep7(acc, ex):
        (g9w,) = ex
        return [_dot(relu(acc), g9w)]

    (u9,) = _sweep(adj_b, u8, [(_bf(g9_w), "full")], [(512, _BF16)], ep7)

    def ep8(acc, ex):
        zh = relu(acc)
        return [zh, zh]

    z_hat, zh_bf = _sweep(adj_b, u9, [], [(512, _F32), (512, _BF16)], ep8)

    adj_hat = _adj_hat(zh_bf)

    return (x_bar, z_hat, adj_hat, q, q1, a_r, z, r, z_l)


# speedup vs baseline: 2.1449x; 1.0084x over previous
import jax
import jax.numpy as jnp
from jax.experimental import pallas as pl
from jax.experimental.pallas import tpu as pltpu

_F32 = jnp.float32
_BF16 = jnp.bfloat16
_N = 4096
_BK = 512
_NK = _N // _BK


def _bf(t):
    return t.astype(_BF16)


def _dot(a, b):
    return jnp.dot(_bf(a), _bf(b), preferred_element_type=_F32)


def _soft_assign_block(h, c_t):
    d2 = (
        jnp.sum(h * h, axis=1, keepdims=True)
        - 2.0 * jnp.dot(h, c_t, preferred_element_type=_F32)
        + jnp.sum(c_t * c_t, axis=0, keepdims=True)
    )
    q = 1.0 / (1.0 + d2)
    return q / jnp.sum(q, axis=1, keepdims=True)


def _sweep(adj, u, extras, out_defs, epilogue, *, bm=512, emit_adj=False):
    m = adj.shape[0]
    nm = m // bm
    du = u.shape[1]
    n_extra = len(extras)

    def body(*refs):
        it = iter(refs)
        adj_ref = next(it)
        u_ref = next(it)
        extra_refs = [next(it) for _ in range(n_extra)]
        outs = [next(it) for _ in range(len(out_defs))]
        adjbf_out = next(it) if emit_adj else None

        a_blk = _bf(adj_ref[...])
        if emit_adj:
            adjbf_out[...] = a_blk
        s = jnp.dot(a_blk, u_ref[...], preferred_element_type=_F32)
        results = epilogue(s, [e[...] for e in extra_refs])
        for o_ref, val in zip(outs, results):
            o_ref[...] = val.astype(o_ref.dtype)

    in_specs = [
        pl.BlockSpec((bm, m), lambda i: (i, 0)),
        pl.BlockSpec((m, du), lambda i: (0, 0)),
    ]
    operands = [adj, u]
    for arr, kind in extras:
        if kind == "row":
            in_specs.append(pl.BlockSpec((bm, arr.shape[1]), lambda i: (i, 0)))
        else:
            in_specs.append(pl.BlockSpec(arr.shape, lambda i: (0, 0)))
        operands.append(arr)

    out_shape = tuple(jax.ShapeDtypeStruct((m, n), dt) for n, dt in out_defs)
    out_specs = tuple(
        pl.BlockSpec((bm, n), lambda i: (i, 0)) for n, _ in out_defs
    )
    if emit_adj:
        out_shape = out_shape + (jax.ShapeDtypeStruct((m, m), _BF16),)
        out_specs = out_specs + (pl.BlockSpec((bm, m), lambda i: (i, 0)),)

    return pl.pallas_call(
        body,
        grid=(nm,),
        in_specs=in_specs,
        out_specs=out_specs,
        out_shape=out_shape,
        compiler_params=pltpu.CompilerParams(
            dimension_semantics=("parallel",),
        ),
    )(*operands)


def _ae_body(
    x_ref,
    e1w, e1b, e2w, e2b, e3w, e3b, zlw, zlb,
    d1w, d1b, d2w, d2b, d3w, d3b, xbw, xbb,
    g1w, ct_ref,
    xbar_o, r_o, re1_o, q1_o, u1_o,
):
    def lin(t, w, b):
        return _dot(t, w[...]) + b[...]

    x = x_ref[...]
    re1 = jnp.maximum(lin(x, e1w, e1b), 0.0)
    re2 = jnp.maximum(lin(re1, e2w, e2b), 0.0)
    re3 = jnp.maximum(lin(re2, e3w, e3b), 0.0)
    r = lin(re3, zlw, zlb)
    rd1 = jnp.maximum(lin(r, d1w, d1b), 0.0)
    rd2 = jnp.maximum(lin(rd1, d2w, d2b), 0.0)
    rd3 = jnp.maximum(lin(rd2, d3w, d3b), 0.0)
    xbar_o[...] = lin(rd3, xbw, xbb)
    r_o[...] = r
    re1_o[...] = _bf(re1)
    q1_o[...] = _soft_assign_block(r, ct_ref[...])
    u1_o[...] = _bf(_dot(x, g1w[...]))


def _ae_branch(x, weights, biases, g1_w, cluster_t, bm=1024):
    m, k = x.shape
    n_in = weights[7].shape[1]
    nz = weights[3].shape[1]
    e1 = weights[0].shape[1]
    nc = cluster_t.shape[1]

    def wspec(w):
        return pl.BlockSpec(w.shape, lambda i: (0, 0))

    operands = [x]
    in_specs = [pl.BlockSpec((bm, k), lambda i: (i, 0))]
    for w, b in zip(weights, biases):
        operands += [w, b]
        in_specs += [wspec(w), wspec(b)]
    operands += [g1_w, cluster_t]
    in_specs += [wspec(g1_w), wspec(cluster_t)]

    def ospec(n):
        return pl.BlockSpec((bm, n), lambda i: (i, 0))

    out_shape = (
        jax.ShapeDtypeStruct((m, n_in), _F32),
        jax.ShapeDtypeStruct((m, nz), _F32),
        jax.ShapeDtypeStruct((m, e1), _BF16),
        jax.ShapeDtypeStruct((m, nc), _F32),
        jax.ShapeDtypeStruct((m, e1), _BF16),
    )
    out_specs = (ospec(n_in), ospec(nz), ospec(e1), ospec(nc), ospec(e1))
    return pl.pallas_call(
        _ae_body,
        grid=(m // bm,),
        in_specs=in_specs,
        out_specs=out_specs,
        out_shape=out_shape,
        compiler_params=pltpu.CompilerParams(
            dimension_semantics=("parallel",),
        ),
    )(*operands)


def _nt_body(a_ref, b_ref, o_ref):
    o = jax.lax.dot_general(
        a_ref[...], b_ref[...], (((1,), (1,)), ((), ())),
        preferred_element_type=_F32,
    )
    o_ref[...] = 0.5 * (1.0 + jnp.tanh(0.5 * o))


def _adj_hat(zh_bf, bm=1024):
    m, k = zh_bf.shape
    return pl.pallas_call(
        _nt_body,
        grid=(m // bm,),
        in_specs=[
            pl.BlockSpec((bm, k), lambda i: (i, 0)),
            pl.BlockSpec((m, k), lambda i: (0, 0)),
        ],
        out_specs=pl.BlockSpec((bm, m), lambda i: (i, 0)),
        out_shape=jax.ShapeDtypeStruct((m, m), _F32),
        compiler_params=pltpu.CompilerParams(
            dimension_semantics=("parallel",),
        ),
    )(zh_bf, zh_bf)


def kernel(x, adj, enc1_w, enc1_b, enc2_w, enc2_b, enc3_w, enc3_b, zl_w, zl_b, dec1_w, dec1_b, dec2_w, dec2_b, dec3_w, dec3_b, xbar_w, xbar_b, g1_w, g2_w, g3_w, g4_w, g5_w, g6_w, g7_w, g8_w, g9_w, cluster):
    row = lambda b: b.reshape(1, -1)
    cluster_t = cluster.T
    nz = cluster.shape[1]
    relu = lambda t: jnp.maximum(t, 0.0)

    x_bar, r, r_e1, q1, u1 = _ae_branch(
        x,
        (_bf(enc1_w), _bf(enc2_w), _bf(enc3_w), _bf(zl_w), _bf(dec1_w),
         _bf(dec2_w), _bf(dec3_w), _bf(xbar_w)),
        (row(enc1_b), row(enc2_b), row(enc3_b), row(zl_b), row(dec1_b),
         row(dec2_b), row(dec3_b), row(xbar_b)),
        _bf(g1_w), cluster_t,
    )

    def ep1(acc, ex):
        re1, e2w, e2b, g2w = ex
        z1 = relu(acc)
        h2 = relu(_dot(z1 + re1.astype(_F32), e2w) + e2b)
        return [h2, _dot(z1, g2w)]

    h2, u2, adj_b = _sweep(
        adj, u1,
        [(r_e1, "row"), (_bf(enc2_w), "full"), (row(enc2_b), "full"),
         (_bf(g2_w), "full")],
        [(500, _BF16), (500, _BF16)], ep1, bm=512, emit_adj=True,
    )

    def ep2(acc, ex):
        (h2v,) = ex
        return [relu(acc) + h2v.astype(_F32)]

    (a3,) = _sweep(adj_b, u2, [(h2, "row")], [(500, _BF16)], ep2)

    def ep3(acc, ex):
        g3w, g4w = ex
        z3 = relu(_dot(acc, g3w))
        return [_dot(z3, g4w)]

    (u4,) = _sweep(adj_b, a3, [(_bf(g3_w), "full"), (_bf(g4_w), "full")],
                   [(nz, _BF16)], ep3)

    pad = lambda t: jnp.pad(t, ((0, 0), (0, 128 - t.shape[1])))

    def ep4(acc, ex):
        (rv,) = ex
        zv = relu(acc)
        return [zv, jnp.concatenate([pad(zv), pad(rv.astype(_F32))], axis=1)]

    z, azr = _sweep(adj_b, u4, [(r, "row")], [(nz, _F32), (256, _BF16)], ep4)


    def ep5(acc, ex):
        g5w, g6w, g7w, ct = ex
        t1 = acc[:, 0:nz]
        zl = t1 + acc[:, 128:128 + nz]
        a_r = relu(_dot(zl, g5w))
        q = _soft_assign_block(zl, ct)
        dec_z1 = relu(_dot(t1, g6w))
        return [a_r, zl, q, _dot(dec_z1, g7w)]

    a_r, z_l, q, u7 = _sweep(
        adj_b, azr,
        [(_bf(g5_w), "full"), (_bf(g6_w), "full"), (_bf(g7_w), "full"),
         (cluster_t, "full")],
        [(nz, _F32), (nz, _F32), (nz, _F32), (500, _BF16)], ep5,
    )

    def ep6(acc, ex):
        (g8w,) = ex
        return [_dot(relu(acc), g8w)]

    (u8,) = _sweep(adj_b, u7, [(_bf(g8_w), "full")], [(500, _BF16)], ep6)

    def ep7(acc, ex):
        (g9w,) = ex
        return [_dot(relu(acc), g9w)]

    (u9,) = _sweep(adj_b, u8, [(_bf(g9_w), "full")], [(512, _BF16)], ep7)

    def ep8(acc, ex):
        zh = relu(acc)
        return [zh, zh]

    z_hat, zh_bf = _sweep(adj_b, u9, [], [(512, _F32), (512, _BF16)], ep8)

    adj_hat = _adj_hat(zh_bf)

    return (x_bar, z_hat, adj_hat, q, q1, a_r, z, r, z_l)


# speedup vs baseline: 2.1811x; 1.0169x over previous
import jax
import jax.numpy as jnp
from jax.experimental import pallas as pl
from jax.experimental.pallas import tpu as pltpu

_F32 = jnp.float32
_BF16 = jnp.bfloat16
_N = 4096
_BK = 512
_NK = _N // _BK


def _bf(t):
    return t.astype(_BF16)


def _dot(a, b):
    return jnp.dot(_bf(a), _bf(b), preferred_element_type=_F32)


def _soft_assign_block(h, c_t):
    d2 = (
        jnp.sum(h * h, axis=1, keepdims=True)
        - 2.0 * jnp.dot(h, c_t, preferred_element_type=_F32)
        + jnp.sum(c_t * c_t, axis=0, keepdims=True)
    )
    q = 1.0 / (1.0 + d2)
    return q / jnp.sum(q, axis=1, keepdims=True)


def _sweep(adj, u, extras, out_defs, epilogue, *, bm=512, emit_adj=False):
    m = adj.shape[0]
    nm = m // bm
    du = u.shape[1]
    n_extra = len(extras)

    def body(*refs):
        it = iter(refs)
        adj_ref = next(it)
        u_ref = next(it)
        extra_refs = [next(it) for _ in range(n_extra)]
        outs = [next(it) for _ in range(len(out_defs))]
        adjbf_out = next(it) if emit_adj else None

        a_blk = _bf(adj_ref[...])
        if emit_adj:
            adjbf_out[...] = a_blk
        s = jnp.dot(a_blk, u_ref[...], preferred_element_type=_F32)
        results = epilogue(s, [e[...] for e in extra_refs])
        for o_ref, val in zip(outs, results):
            o_ref[...] = val.astype(o_ref.dtype)

    in_specs = [
        pl.BlockSpec((bm, m), lambda i: (i, 0)),
        pl.BlockSpec((m, du), lambda i: (0, 0)),
    ]
    operands = [adj, u]
    for arr, kind in extras:
        if kind == "row":
            in_specs.append(pl.BlockSpec((bm, arr.shape[1]), lambda i: (i, 0)))
        else:
            in_specs.append(pl.BlockSpec(arr.shape, lambda i: (0, 0)))
        operands.append(arr)

    out_shape = tuple(jax.ShapeDtypeStruct((m, n), dt) for n, dt in out_defs)
    out_specs = tuple(
        pl.BlockSpec((bm, n), lambda i: (i, 0)) for n, _ in out_defs
    )
    if emit_adj:
        out_shape = out_shape + (jax.ShapeDtypeStruct((m, m), _BF16),)
        out_specs = out_specs + (pl.BlockSpec((bm, m), lambda i: (i, 0)),)

    return pl.pallas_call(
        body,
        grid=(nm,),
        in_specs=in_specs,
        out_specs=out_specs,
        out_shape=out_shape,
        compiler_params=pltpu.CompilerParams(
            dimension_semantics=("parallel",),
        ),
    )(*operands)


def _mega_body(
    adj_ref, u2_ref, h2_ref, r_ref,
    g3w, g4w, g5w, g6w, g7w, g8w, g9w, ct_ref,
    z_o, ar_o, zl_o, q_o, zhat_o, zhbf_o,
    ua, ub,
    *, bm, nz,
):
    s = pl.program_id(0)
    mi = pl.program_id(1)
    rows = pl.ds(mi * bm, bm)
    relu = lambda t: jnp.maximum(t, 0.0)
    a_blk = adj_ref[...]

    def mm(u_val):
        return jnp.dot(a_blk, u_val, preferred_element_type=_F32)

    pad128 = lambda t: jnp.pad(t, ((0, 0), (0, 128 - t.shape[1])))
    pad512 = lambda t: jnp.pad(t, ((0, 0), (0, 512 - t.shape[1])))

    @pl.when(s == 0)
    def _s0():
        sv = mm(u2_ref[...])
        ua[rows, :] = _bf(relu(sv) + h2_ref[...].astype(_F32))

    @pl.when(s == 1)
    def _s1():
        sv = mm(ua[...])
        z3 = relu(_dot(sv, g3w[...]))
        ub[rows, :] = _bf(pad512(_dot(z3, g4w[...])))

    @pl.when(s == 2)
    def _s2():
        sv = mm(ub[:, :128])
        zv = relu(sv)
        z_o[rows, :] = zv[:, :nz]
        rv = r_ref[rows, :].astype(_F32)
        ua[rows, :] = _bf(
            jnp.concatenate(
                [zv, pad128(rv), jnp.zeros((bm, 256), _F32)], axis=1
            )
        )

    @pl.when(s == 3)
    def _s3():
        sv = mm(ua[:, :256])
        t1 = sv[:, :nz]
        zl = t1 + sv[:, 128:128 + nz]
        zl_o[rows, :] = zl
        q_o[rows, :] = _soft_assign_block(zl, ct_ref[...])
        ar_o[rows, :] = relu(_dot(zl, g5w[...]))
        dec_z1 = relu(_dot(t1, g6w[...]))
        ub[rows, :] = _bf(pad512(_dot(dec_z1, g7w[...])))

    @pl.when(s == 4)
    def _s4():
        sv = mm(ub[...])
        ua[rows, :] = _bf(pad512(_dot(relu(sv), g8w[...])))

    @pl.when(s == 5)
    def _s5():
        sv = mm(ua[...])
        ub[rows, :] = _bf(_dot(relu(sv), g9w[...]))

    @pl.when(s == 6)
    def _s6():
        sv = mm(ub[...])
        zh = relu(sv)
        zhat_o[rows, :] = zh
        zhbf_o[rows, :] = _bf(zh)


def _gnn_mega(adj_b, u2p, h2p, r, g3w, g4w, g5w, g6w, g7w, g8w, g9w,
              cluster_t, bm=512):
    m = adj_b.shape[0]
    nz = r.shape[1]
    import functools as _ft

    full = lambda arr: pl.BlockSpec(arr.shape, lambda s, i: (0, 0))
    in_specs = [
        pl.BlockSpec((bm, m), lambda s, i: (i, 0)),
        full(u2p),
        pl.BlockSpec((bm, 512), lambda s, i: (i, 0)),
        full(r),
        full(g3w), full(g4w), full(g5w), full(g6w), full(g7w), full(g8w),
        full(g9w), full(cluster_t),
    ]
    out_shape = (
        jax.ShapeDtypeStruct((m, nz), _F32),
        jax.ShapeDtypeStruct((m, nz), _F32),
        jax.ShapeDtypeStruct((m, nz), _F32),
        jax.ShapeDtypeStruct((m, nz), _F32),
        jax.ShapeDtypeStruct((m, 512), _F32),
        jax.ShapeDtypeStruct((m, 512), _BF16),
    )
    out_specs = tuple(
        pl.BlockSpec(sh.shape, lambda s, i: (0, 0)) for sh in out_shape
    )
    return pl.pallas_call(
        _ft.partial(_mega_body, bm=bm, nz=nz),
        grid=(7, m // bm),
        in_specs=in_specs,
        out_specs=out_specs,
        out_shape=out_shape,
        scratch_shapes=[
            pltpu.VMEM((m, 512), _BF16),
            pltpu.VMEM((m, 512), _BF16),
        ],
        compiler_params=pltpu.CompilerParams(
            dimension_semantics=("arbitrary", "arbitrary"),
        ),
    )(adj_b, u2p, h2p, r, g3w, g4w, g5w, g6w, g7w, g8w, g9w, cluster_t)


def _ae_body(
    x_ref,
    e1w, e1b, e2w, e2b, e3w, e3b, zlw, zlb,
    d1w, d1b, d2w, d2b, d3w, d3b, xbw, xbb,
    g1w, ct_ref,
    xbar_o, r_o, re1_o, q1_o, u1_o,
):
    def lin(t, w, b):
        return _dot(t, w[...]) + b[...]

    x = x_ref[...]
    re1 = jnp.maximum(lin(x, e1w, e1b), 0.0)
    re2 = jnp.maximum(lin(re1, e2w, e2b), 0.0)
    re3 = jnp.maximum(lin(re2, e3w, e3b), 0.0)
    r = lin(re3, zlw, zlb)
    rd1 = jnp.maximum(lin(r, d1w, d1b), 0.0)
    rd2 = jnp.maximum(lin(rd1, d2w, d2b), 0.0)
    rd3 = jnp.maximum(lin(rd2, d3w, d3b), 0.0)
    xbar_o[...] = lin(rd3, xbw, xbb)
    r_o[...] = r
    re1_o[...] = _bf(re1)
    q1_o[...] = _soft_assign_block(r, ct_ref[...])
    u1_o[...] = _bf(_dot(x, g1w[...]))


def _ae_branch(x, weights, biases, g1_w, cluster_t, bm=1024):
    m, k = x.shape
    n_in = weights[7].shape[1]
    nz = weights[3].shape[1]
    e1 = weights[0].shape[1]
    nc = cluster_t.shape[1]

    def wspec(w):
        return pl.BlockSpec(w.shape, lambda i: (0, 0))

    operands = [x]
    in_specs = [pl.BlockSpec((bm, k), lambda i: (i, 0))]
    for w, b in zip(weights, biases):
        operands += [w, b]
        in_specs += [wspec(w), wspec(b)]
    operands += [g1_w, cluster_t]
    in_specs += [wspec(g1_w), wspec(cluster_t)]

    def ospec(n):
        return pl.BlockSpec((bm, n), lambda i: (i, 0))

    out_shape = (
        jax.ShapeDtypeStruct((m, n_in), _F32),
        jax.ShapeDtypeStruct((m, nz), _F32),
        jax.ShapeDtypeStruct((m, e1), _BF16),
        jax.ShapeDtypeStruct((m, nc), _F32),
        jax.ShapeDtypeStruct((m, e1), _BF16),
    )
    out_specs = (ospec(n_in), ospec(nz), ospec(e1), ospec(nc), ospec(e1))
    return pl.pallas_call(
        _ae_body,
        grid=(m // bm,),
        in_specs=in_specs,
        out_specs=out_specs,
        out_shape=out_shape,
        compiler_params=pltpu.CompilerParams(
            dimension_semantics=("parallel",),
        ),
    )(*operands)


def _nt_body(a_ref, b_ref, o_ref):
    o = jax.lax.dot_general(
        a_ref[...], b_ref[...], (((1,), (1,)), ((), ())),
        preferred_element_type=_F32,
    )
    o_ref[...] = 0.5 * (1.0 + jnp.tanh(0.5 * o))


def _adj_hat(zh_bf, bm=1024):
    m, k = zh_bf.shape
    return pl.pallas_call(
        _nt_body,
        grid=(m // bm,),
        in_specs=[
            pl.BlockSpec((bm, k), lambda i: (i, 0)),
            pl.BlockSpec((m, k), lambda i: (0, 0)),
        ],
        out_specs=pl.BlockSpec((bm, m), lambda i: (i, 0)),
        out_shape=jax.ShapeDtypeStruct((m, m), _F32),
        compiler_params=pltpu.CompilerParams(
            dimension_semantics=("parallel",),
        ),
    )(zh_bf, zh_bf)


def kernel(x, adj, enc1_w, enc1_b, enc2_w, enc2_b, enc3_w, enc3_b, zl_w, zl_b, dec1_w, dec1_b, dec2_w, dec2_b, dec3_w, dec3_b, xbar_w, xbar_b, g1_w, g2_w, g3_w, g4_w, g5_w, g6_w, g7_w, g8_w, g9_w, cluster):
    row = lambda b: b.reshape(1, -1)
    cluster_t = cluster.T
    nz = cluster.shape[1]
    relu = lambda t: jnp.maximum(t, 0.0)

    x_bar, r, r_e1, q1, u1 = _ae_branch(
        x,
        (_bf(enc1_w), _bf(enc2_w), _bf(enc3_w), _bf(zl_w), _bf(dec1_w),
         _bf(dec2_w), _bf(dec3_w), _bf(xbar_w)),
        (row(enc1_b), row(enc2_b), row(enc3_b), row(zl_b), row(dec1_b),
         row(dec2_b), row(dec3_b), row(xbar_b)),
        _bf(g1_w), cluster_t,
    )

    pad512 = lambda t: jnp.pad(t, ((0, 0), (0, 512 - t.shape[1])))

    def ep1(acc, ex):
        re1, e2w, e2b, g2w = ex
        z1 = relu(acc)
        h2 = relu(_dot(z1 + re1.astype(_F32), e2w) + e2b)
        return [pad512(h2), pad512(_dot(z1, g2w))]

    h2p, u2p, adj_b = _sweep(
        adj, u1,
        [(r_e1, "row"), (_bf(enc2_w), "full"), (row(enc2_b), "full"),
         (_bf(g2_w), "full")],
        [(512, _BF16), (512, _BF16)], ep1, bm=512, emit_adj=True,
    )

    padr = lambda t, n: jnp.pad(t, ((0, n - t.shape[0]), (0, 0)))
    z, a_r, z_l, q, z_hat, zh_bf = _gnn_mega(
        adj_b, u2p, h2p, r,
        _bf(padr(g3_w, 512)), _bf(g4_w), _bf(g5_w), _bf(g6_w), _bf(g7_w),
        _bf(padr(g8_w, 512)), _bf(padr(g9_w, 512)), cluster_t,
    )

    adj_hat = _adj_hat(zh_bf)

    return (x_bar, z_hat, adj_hat, q, q1, a_r, z, r, z_l)


# speedup vs baseline: 2.1840x; 1.0013x over previous
import jax
import jax.numpy as jnp
from jax.experimental import pallas as pl
from jax.experimental.pallas import tpu as pltpu

_F32 = jnp.float32
_BF16 = jnp.bfloat16
_N = 4096
_BK = 512
_NK = _N // _BK


def _bf(t):
    return t.astype(_BF16)


def _dot(a, b):
    return jnp.dot(_bf(a), _bf(b), preferred_element_type=_F32)


def _soft_assign_block(h, c_t):
    d2 = (
        jnp.sum(h * h, axis=1, keepdims=True)
        - 2.0 * jnp.dot(h, c_t, preferred_element_type=_F32)
        + jnp.sum(c_t * c_t, axis=0, keepdims=True)
    )
    q = 1.0 / (1.0 + d2)
    return q / jnp.sum(q, axis=1, keepdims=True)


def _sweep(adj, u, extras, out_defs, epilogue, *, bm=512, emit_adj=False):
    m = adj.shape[0]
    nm = m // bm
    du = u.shape[1]
    n_extra = len(extras)

    def body(*refs):
        it = iter(refs)
        adj_ref = next(it)
        u_ref = next(it)
        extra_refs = [next(it) for _ in range(n_extra)]
        outs = [next(it) for _ in range(len(out_defs))]
        adjbf_out = next(it) if emit_adj else None

        a_blk = _bf(adj_ref[...])
        if emit_adj:
            adjbf_out[...] = a_blk
        s = jnp.dot(a_blk, u_ref[...], preferred_element_type=_F32)
        results = epilogue(s, [e[...] for e in extra_refs])
        for o_ref, val in zip(outs, results):
            o_ref[...] = val.astype(o_ref.dtype)

    in_specs = [
        pl.BlockSpec((bm, m), lambda i: (i, 0)),
        pl.BlockSpec((m, du), lambda i: (0, 0)),
    ]
    operands = [adj, u]
    for arr, kind in extras:
        if kind == "row":
            in_specs.append(pl.BlockSpec((bm, arr.shape[1]), lambda i: (i, 0)))
        else:
            in_specs.append(pl.BlockSpec(arr.shape, lambda i: (0, 0)))
        operands.append(arr)

    out_shape = tuple(jax.ShapeDtypeStruct((m, n), dt) for n, dt in out_defs)
    out_specs = tuple(
        pl.BlockSpec((bm, n), lambda i: (i, 0)) for n, _ in out_defs
    )
    if emit_adj:
        out_shape = out_shape + (jax.ShapeDtypeStruct((m, m), _BF16),)
        out_specs = out_specs + (pl.BlockSpec((bm, m), lambda i: (i, 0)),)

    return pl.pallas_call(
        body,
        grid=(nm,),
        in_specs=in_specs,
        out_specs=out_specs,
        out_shape=out_shape,
        compiler_params=pltpu.CompilerParams(
            dimension_semantics=("parallel",),
        ),
    )(*operands)


def _mega_body(
    adj_ref, u2_ref, h2_ref, r_ref,
    g3w, g4w, g5w, g6w, g7w, g8w, g9w, ct_ref,
    z_o, ar_o, zl_o, q_o, zhat_o, zhbf_o,
    ua, ub,
    *, bm, nz,
):
    s = pl.program_id(0)
    mi = pl.program_id(1)
    rows = pl.ds(mi * bm, bm)
    relu = lambda t: jnp.maximum(t, 0.0)
    a_blk = adj_ref[...]

    def mm(u_val):
        return jnp.dot(a_blk, u_val, preferred_element_type=_F32)

    pad128 = lambda t: jnp.pad(t, ((0, 0), (0, 128 - t.shape[1])))
    pad512 = lambda t: jnp.pad(t, ((0, 0), (0, 512 - t.shape[1])))

    @pl.when(s == 0)
    def _s0():
        sv = mm(u2_ref[...])
        ua[rows, :] = _bf(relu(sv) + h2_ref[...].astype(_F32))

    @pl.when(s == 1)
    def _s1():
        sv = mm(ua[...])
        z3 = relu(_dot(sv, g3w[...]))
        ub[rows, :] = _bf(pad512(_dot(z3, g4w[...])))

    @pl.when(s == 2)
    def _s2():
        sv = mm(ub[:, :128])
        zv = relu(sv)
        z_o[rows, :] = zv[:, :nz]
        rv = r_ref[rows, :].astype(_F32)
        ua[rows, :] = _bf(
            jnp.concatenate(
                [zv, pad128(rv), jnp.zeros((bm, 256), _F32)], axis=1
            )
        )

    @pl.when(s == 3)
    def _s3():
        sv = mm(ua[:, :256])
        t1 = sv[:, :nz]
        zl = t1 + sv[:, 128:128 + nz]
        zl_o[rows, :] = zl
        q_o[rows, :] = _soft_assign_block(zl, ct_ref[...])
        ar_o[rows, :] = relu(_dot(zl, g5w[...]))
        dec_z1 = relu(_dot(t1, g6w[...]))
        ub[rows, :] = _bf(pad512(_dot(dec_z1, g7w[...])))

    @pl.when(s == 4)
    def _s4():
        sv = mm(ub[...])
        ua[rows, :] = _bf(pad512(_dot(relu(sv), g8w[...])))

    @pl.when(s == 5)
    def _s5():
        sv = mm(ua[...])
        ub[rows, :] = _bf(_dot(relu(sv), g9w[...]))

    @pl.when(s == 6)
    def _s6():
        sv = mm(ub[...])
        zh = relu(sv)
        zhat_o[rows, :] = zh
        zhbf_o[rows, :] = _bf(zh)


def _gnn_mega(adj_b, u2p, h2p, r, g3w, g4w, g5w, g6w, g7w, g8w, g9w,
              cluster_t, bm=512):
    m = adj_b.shape[0]
    nz = r.shape[1]
    import functools as _ft

    full = lambda arr: pl.BlockSpec(arr.shape, lambda s, i: (0, 0))
    in_specs = [
        pl.BlockSpec((bm, m), lambda s, i: (i, 0)),
        full(u2p),
        pl.BlockSpec((bm, 512), lambda s, i: (i, 0)),
        full(r),
        full(g3w), full(g4w), full(g5w), full(g6w), full(g7w), full(g8w),
        full(g9w), full(cluster_t),
    ]
    out_shape = (
        jax.ShapeDtypeStruct((m, nz), _F32),
        jax.ShapeDtypeStruct((m, nz), _F32),
        jax.ShapeDtypeStruct((m, nz), _F32),
        jax.ShapeDtypeStruct((m, nz), _F32),
        jax.ShapeDtypeStruct((m, 512), _F32),
        jax.ShapeDtypeStruct((m, 512), _BF16),
    )
    out_specs = tuple(
        pl.BlockSpec(sh.shape, lambda s, i: (0, 0)) for sh in out_shape
    )
    return pl.pallas_call(
        _ft.partial(_mega_body, bm=bm, nz=nz),
        grid=(7, m // bm),
        in_specs=in_specs,
        out_specs=out_specs,
        out_shape=out_shape,
        scratch_shapes=[
            pltpu.VMEM((m, 512), _BF16),
            pltpu.VMEM((m, 512), _BF16),
        ],
        compiler_params=pltpu.CompilerParams(
            dimension_semantics=("arbitrary", "arbitrary"),
        ),
    )(adj_b, u2p, h2p, r, g3w, g4w, g5w, g6w, g7w, g8w, g9w, cluster_t)


def _ae_body(
    x_ref,
    e1w, e1b, e2w, e2b, e3w, e3b, zlw, zlb,
    d1w, d1b, d2w, d2b, d3w, d3b, xbw, xbb,
    g1w, ct_ref,
    xbar_o, r_o, re1_o, q1_o, u1_o,
):
    def lin(t, w, b):
        return jnp.dot(t, w[...], preferred_element_type=_F32) + b[...]

    x = x_ref[...]
    re1 = jnp.maximum(lin(x, e1w, e1b), 0.0)
    re2 = jnp.maximum(lin(re1, e2w, e2b), 0.0)
    re3 = jnp.maximum(lin(re2, e3w, e3b), 0.0)
    r = lin(re3, zlw, zlb)
    rd1 = jnp.maximum(lin(r, d1w, d1b), 0.0)
    rd2 = jnp.maximum(lin(rd1, d2w, d2b), 0.0)
    rd3 = jnp.maximum(lin(rd2, d3w, d3b), 0.0)
    xbar_o[...] = lin(rd3, xbw, xbb)
    r_o[...] = r
    re1_o[...] = _bf(re1)
    q1_o[...] = _soft_assign_block(r, ct_ref[...])
    u1_o[...] = _bf(_dot(x, g1w[...]))


def _ae_branch(x, weights, biases, g1_w, cluster_t, bm=1024):
    m, k = x.shape
    n_in = weights[7].shape[1]
    nz = weights[3].shape[1]
    e1 = weights[0].shape[1]
    nc = cluster_t.shape[1]

    def wspec(w):
        return pl.BlockSpec(w.shape, lambda i: (0, 0))

    operands = [x]
    in_specs = [pl.BlockSpec((bm, k), lambda i: (i, 0))]
    for w, b in zip(weights, biases):
        operands += [w, b]
        in_specs += [wspec(w), wspec(b)]
    operands += [g1_w, cluster_t]
    in_specs += [wspec(g1_w), wspec(cluster_t)]

    def ospec(n):
        return pl.BlockSpec((bm, n), lambda i: (i, 0))

    out_shape = (
        jax.ShapeDtypeStruct((m, n_in), _F32),
        jax.ShapeDtypeStruct((m, nz), _F32),
        jax.ShapeDtypeStruct((m, e1), _BF16),
        jax.ShapeDtypeStruct((m, nc), _F32),
        jax.ShapeDtypeStruct((m, e1), _BF16),
    )
    out_specs = (ospec(n_in), ospec(nz), ospec(e1), ospec(nc), ospec(e1))
    return pl.pallas_call(
        _ae_body,
        grid=(m // bm,),
        in_specs=in_specs,
        out_specs=out_specs,
        out_shape=out_shape,
        compiler_params=pltpu.CompilerParams(
            dimension_semantics=("parallel",),
        ),
    )(*operands)


def _nt_body(a_ref, b_ref, o_ref):
    o = jax.lax.dot_general(
        a_ref[...], b_ref[...], (((1,), (1,)), ((), ())),
        preferred_element_type=_F32,
    )
    o_ref[...] = 0.5 * (1.0 + jnp.tanh(0.5 * o))


def _adj_hat(zh_bf, bm=1024):
    m, k = zh_bf.shape
    return pl.pallas_call(
        _nt_body,
        grid=(m // bm,),
        in_specs=[
            pl.BlockSpec((bm, k), lambda i: (i, 0)),
            pl.BlockSpec((m, k), lambda i: (0, 0)),
        ],
        out_specs=pl.BlockSpec((bm, m), lambda i: (i, 0)),
        out_shape=jax.ShapeDtypeStruct((m, m), _F32),
        compiler_params=pltpu.CompilerParams(
            dimension_semantics=("parallel",),
        ),
    )(zh_bf, zh_bf)


def kernel(x, adj, enc1_w, enc1_b, enc2_w, enc2_b, enc3_w, enc3_b, zl_w, zl_b, dec1_w, dec1_b, dec2_w, dec2_b, dec3_w, dec3_b, xbar_w, xbar_b, g1_w, g2_w, g3_w, g4_w, g5_w, g6_w, g7_w, g8_w, g9_w, cluster):
    row = lambda b: b.reshape(1, -1)
    cluster_t = cluster.T
    nz = cluster.shape[1]
    relu = lambda t: jnp.maximum(t, 0.0)

    x_bar, r, r_e1, q1, u1 = _ae_branch(
        x,
        (enc1_w, enc2_w, enc3_w, zl_w, dec1_w, dec2_w, dec3_w, xbar_w),
        (row(enc1_b), row(enc2_b), row(enc3_b), row(zl_b), row(dec1_b),
         row(dec2_b), row(dec3_b), row(xbar_b)),
        _bf(g1_w), cluster_t,
    )

    pad512 = lambda t: jnp.pad(t, ((0, 0), (0, 512 - t.shape[1])))

    def ep1(acc, ex):
        re1, e2w, e2b, g2w = ex
        z1 = relu(acc)
        h2 = relu(_dot(z1 + re1.astype(_F32), e2w) + e2b)
        return [pad512(h2), pad512(_dot(z1, g2w))]

    h2p, u2p, adj_b = _sweep(
        adj, u1,
        [(r_e1, "row"), (_bf(enc2_w), "full"), (row(enc2_b), "full"),
         (_bf(g2_w), "full")],
        [(512, _BF16), (512, _BF16)], ep1, bm=512, emit_adj=True,
    )

    padr = lambda t, n: jnp.pad(t, ((0, n - t.shape[0]), (0, 0)))
    z, a_r, z_l, q, z_hat, zh_bf = _gnn_mega(
        adj_b, u2p, h2p, r,
        _bf(padr(g3_w, 512)), _bf(g4_w), _bf(g5_w), _bf(g6_w), _bf(g7_w),
        _bf(padr(g8_w, 512)), _bf(padr(g9_w, 512)), cluster_t,
    )

    adj_hat = _adj_hat(zh_bf)

    return (x_bar, z_hat, adj_hat, q, q1, a_r, z, r, z_l)
